# baseline (speedup 1.0000x reference)
import functools

import jax
import jax.numpy as jnp
from jax import lax
from jax.experimental import pallas as pl
from jax.experimental.pallas import tpu as pltpu
from jax.experimental.pallas import tpu_sc as plsc

N = 10000
E = 320000
D = 128
EPS = 1e-5

NP = 10240
CHUNK = 128
NUM_WORKERS = 32
EPT = 10240
EPAD = EPT * NUM_WORKERS
NCHUNK = EPT // CHUNK
CH_A = 64
NCH_A = EPT // CH_A
SBLK = 40
NSTG = NCH_A // SBLK
NBUF = 4
ROWS_PT = NP // 16

_mesh = plsc.VectorSubcoreMesh(core_axis_name="c", subcore_axis_name="s")


@functools.partial(
    pl.kernel,
    out_type=jax.ShapeDtypeStruct((2, NP, D), jnp.float32),
    mesh=_mesh,
    scratch_types=[
        pltpu.VMEM((NCHUNK, CHUNK), jnp.int32),
        pltpu.VMEM((CHUNK, D), jnp.float32),
        pltpu.VMEM((16, D), jnp.float32),
        pltpu.VMEM_SHARED((NP, D), jnp.float32),
    ],
)
def _deg_kernel(dst_hbm, out_hbm, didx_all, ones_v, zero_v, acc):
    c = lax.axis_index("c")
    s = lax.axis_index("s")
    wid = s * 2 + c

    pltpu.sync_copy(dst_hbm.at[pl.ds(wid * NCHUNK, NCHUNK)], didx_all)

    for i in range(16):
        for j in range(D // 16):
            zero_v[i, pl.ds(j * 16, 16)] = jnp.zeros((16,), jnp.float32)
    for i in range(CHUNK):
        for j in range(D // 16):
            ones_v[i, pl.ds(j * 16, 16)] = jnp.ones((16,), jnp.float32)

    for k in range(ROWS_PT // 16):
        pltpu.sync_copy(zero_v, acc.at[pl.ds(s * ROWS_PT + k * 16, 16)])
    plsc.subcore_barrier()

    def body(g, carry):
        pltpu.sync_copy(ones_v, acc.at[didx_all.at[g]], add=True)
        return carry

    lax.fori_loop(0, NCHUNK, body, 0)
    plsc.subcore_barrier()

    pltpu.sync_copy(
        acc.at[pl.ds(s * ROWS_PT, ROWS_PT)],
        out_hbm.at[c, pl.ds(s * ROWS_PT, ROWS_PT)],
    )


@functools.partial(
    pl.kernel,
    out_type=jax.ShapeDtypeStruct((2, NP, D), jnp.float32),
    mesh=_mesh,
    scratch_types=[
        pltpu.VMEM((SBLK, CH_A), jnp.int32),
        pltpu.VMEM((SBLK, CH_A), jnp.int32),
        pltpu.VMEM((NBUF, CH_A, D), jnp.float32),
        pltpu.VMEM((16, D), jnp.float32),
        pltpu.VMEM_SHARED((NP, D), jnp.float32),
    ] + [pltpu.SemaphoreType.DMA] * NBUF,
)
def _agg_kernel(ht_hbm, src_hbm, dst_hbm, out_hbm, sidx, didx,
                rows, zero_v, acc, *sems):
    c = lax.axis_index("c")
    s = lax.axis_index("s")
    wid = s * 2 + c

    for i in range(16):
        for j in range(D // 16):
            zero_v[i, pl.ds(j * 16, 16)] = jnp.zeros((16,), jnp.float32)

    for k in range(ROWS_PT // 16):
        pltpu.sync_copy(zero_v, acc.at[pl.ds(s * ROWS_PT + k * 16, 16)])
    plsc.subcore_barrier()

    def stage(t, carry):
        base = wid * NCH_A + t * SBLK
        pltpu.sync_copy(src_hbm.at[pl.ds(base, SBLK)], sidx)
        pltpu.sync_copy(dst_hbm.at[pl.ds(base, SBLK)], didx)

        for j in range(NBUF):
            pltpu.async_copy(ht_hbm.at[sidx.at[j]], rows.at[j], sems[j])

        def blk(b, c2):
            bb = b * NBUF
            for j in range(NBUF):
                g = bb + j
                pltpu.make_async_copy(ht_hbm.at[sidx.at[g]], rows.at[j], sems[j]).wait()
                pltpu.sync_copy(rows.at[j], acc.at[didx.at[g]], add=True)
                pltpu.async_copy(ht_hbm.at[sidx.at[g + NBUF]], rows.at[j], sems[j])
            return c2

        lax.fori_loop(0, SBLK // NBUF - 1, blk, 0)

        bb = (SBLK // NBUF - 1) * NBUF
        for j in range(NBUF):
            g = bb + j
            pltpu.make_async_copy(ht_hbm.at[sidx.at[g]], rows.at[j], sems[j]).wait()
            pltpu.sync_copy(rows.at[j], acc.at[didx.at[g]], add=True)
        return carry

    lax.fori_loop(0, NSTG, stage, 0)
    plsc.subcore_barrier()

    pltpu.sync_copy(
        acc.at[pl.ds(s * ROWS_PT, ROWS_PT)],
        out_hbm.at[c, pl.ds(s * ROWS_PT, ROWS_PT)],
    )


def _t1_body(xp_ref, w1_ref, cnt_ref, ht_ref, dinv_ref):
    cnt = cnt_ref[0, :, 0:1] + cnt_ref[1, :, 0:1]
    deg = cnt + 1.0
    row = lax.broadcasted_iota(jnp.int32, (NP, 1), 0)
    dinv = jnp.where(row < N, lax.rsqrt(deg), 0.0)
    ht = jnp.dot(xp_ref[...], w1_ref[...], preferred_element_type=jnp.float32)
    ht_ref[...] = ht * dinv
    dinv_ref[...] = dinv


def _t2_body(p_ref, ht1_ref, dinv_ref, b1_ref, g_ref, be_ref, w2_ref, ht2_ref):
    agg = p_ref[0] + p_ref[1] + ht1_ref[...]
    o1 = agg * dinv_ref[...] + b1_ref[...]
    o1r = o1[:N]
    mean = jnp.mean(o1r, axis=0, keepdims=True)
    var = jnp.mean((o1r - mean) ** 2, axis=0, keepdims=True)
    h2 = jnp.maximum((o1 - mean) * lax.rsqrt(var + EPS) * g_ref[...] + be_ref[...], 0.0)
    ht2 = jnp.dot(h2, w2_ref[...], preferred_element_type=jnp.float32)
    ht2_ref[...] = ht2 * dinv_ref[...]


def _t3_body(p_ref, ht2_ref, dinv_ref, b2_ref, out_ref):
    agg = p_ref[0, :N] + p_ref[1, :N] + ht2_ref[:N]
    out_ref[...] = agg * dinv_ref[:N] + b2_ref[...]


_t1 = pl.pallas_call(
    _t1_body,
    out_shape=(
        jax.ShapeDtypeStruct((NP, D), jnp.float32),
        jax.ShapeDtypeStruct((NP, 1), jnp.float32),
    ),
)

_t2 = pl.pallas_call(
    _t2_body,
    out_shape=jax.ShapeDtypeStruct((NP, D), jnp.float32),
)

_t3 = pl.pallas_call(
    _t3_body,
    out_shape=jax.ShapeDtypeStruct((N, D), jnp.float32),
)


def kernel(x, edge_index, W1, b1, gamma, beta, W2, b2):
    src = edge_index[0]
    dst = edge_index[1]
    pad = jnp.full((EPAD - E,), N, dtype=jnp.int32)
    srcf = jnp.concatenate([src, pad])
    dstf = jnp.concatenate([dst, pad])
    dstp = dstf.reshape(EPAD // CHUNK, CHUNK)
    srca = srcf.reshape(EPAD // CH_A, CH_A)
    dsta = dstf.reshape(EPAD // CH_A, CH_A)
    xp = jnp.pad(x, ((0, NP - N), (0, 0)))

    cnt = _deg_kernel(dstp)
    ht1, dinv = _t1(xp, W1, cnt)
    p1 = _agg_kernel(ht1, srca, dsta)
    ht2 = _t2(p1, ht1, dinv, b1.reshape(1, D), gamma.reshape(1, D),
              beta.reshape(1, D), W2)
    p2 = _agg_kernel(ht2, srca, dsta)
    out = _t3(p2, ht2, dinv, b2.reshape(1, D))
    return out


# speedup vs baseline: 1.0433x; 1.0433x over previous
import functools

import jax
import jax.numpy as jnp
from jax import lax
from jax.experimental import pallas as pl
from jax.experimental.pallas import tpu as pltpu
from jax.experimental.pallas import tpu_sc as plsc

N = 10000
E = 320000
D = 128
EPS = 1e-5

NP = 10240
CHUNK = 128
NUM_WORKERS = 32
EPT = 10240
EPAD = EPT * NUM_WORKERS
NCHUNK = EPT // CHUNK
IDXBLK = 8
ROWS_PT = NP // 16

_mesh = plsc.VectorSubcoreMesh(core_axis_name="c", subcore_axis_name="s")


@functools.partial(
    pl.kernel,
    out_type=jax.ShapeDtypeStruct((2, NP, D), jnp.float32),
    mesh=_mesh,
    scratch_types=[
        pltpu.VMEM((NCHUNK, CHUNK), jnp.int32),
        pltpu.VMEM((CHUNK, D), jnp.float32),
        pltpu.VMEM((16, D), jnp.float32),
        pltpu.VMEM_SHARED((NP, D), jnp.float32),
    ],
)
def _deg_kernel(dst_hbm, out_hbm, didx_all, ones_v, zero_v, acc):
    c = lax.axis_index("c")
    s = lax.axis_index("s")
    wid = s * 2 + c

    pltpu.sync_copy(dst_hbm.at[pl.ds(wid * NCHUNK, NCHUNK)], didx_all)

    for i in range(16):
        for j in range(D // 16):
            zero_v[i, pl.ds(j * 16, 16)] = jnp.zeros((16,), jnp.float32)
    for i in range(CHUNK):
        for j in range(D // 16):
            ones_v[i, pl.ds(j * 16, 16)] = jnp.ones((16,), jnp.float32)

    for k in range(ROWS_PT // 16):
        pltpu.sync_copy(zero_v, acc.at[pl.ds(s * ROWS_PT + k * 16, 16)])
    plsc.subcore_barrier()

    def body(g, carry):
        pltpu.sync_copy(ones_v, acc.at[didx_all.at[g]], add=True)
        return carry

    lax.fori_loop(0, NCHUNK, body, 0)
    plsc.subcore_barrier()

    pltpu.sync_copy(
        acc.at[pl.ds(s * ROWS_PT, ROWS_PT)],
        out_hbm.at[c, pl.ds(s * ROWS_PT, ROWS_PT)],
    )


@functools.partial(
    pl.kernel,
    out_type=jax.ShapeDtypeStruct((2, NP, D), jnp.float32),
    mesh=_mesh,
    scratch_types=[
        pltpu.VMEM((IDXBLK, CHUNK), jnp.int32),
        pltpu.VMEM((IDXBLK, CHUNK), jnp.int32),
        pltpu.VMEM((CHUNK, D), jnp.float32),
        pltpu.VMEM((CHUNK, D), jnp.float32),
        pltpu.VMEM((16, D), jnp.float32),
        pltpu.VMEM_SHARED((NP, D), jnp.float32),
        pltpu.SemaphoreType.DMA,
        pltpu.SemaphoreType.DMA,
    ],
)
def _agg_kernel(ht_hbm, src_hbm, dst_hbm, out_hbm, sidx_blk, didx_blk,
                rows0, rows1, zero_v, acc, sem0, sem1):
    c = lax.axis_index("c")
    s = lax.axis_index("s")
    wid = s * 2 + c

    for i in range(16):
        for j in range(D // 16):
            zero_v[i, pl.ds(j * 16, 16)] = jnp.zeros((16,), jnp.float32)

    for k in range(ROWS_PT // 16):
        pltpu.sync_copy(zero_v, acc.at[pl.ds(s * ROWS_PT + k * 16, 16)])
    plsc.subcore_barrier()

    def body(b, carry):
        base = wid * NCHUNK + b * IDXBLK
        pltpu.sync_copy(src_hbm.at[pl.ds(base, IDXBLK)], sidx_blk)
        pltpu.sync_copy(dst_hbm.at[pl.ds(base, IDXBLK)], didx_blk)
        pltpu.async_copy(ht_hbm.at[sidx_blk.at[0]], rows0, sem0)
        for j in range(IDXBLK):
            rows, sem = (rows0, sem0) if j % 2 == 0 else (rows1, sem1)
            nrows, nsem = (rows1, sem1) if j % 2 == 0 else (rows0, sem0)
            pltpu.make_async_copy(ht_hbm.at[sidx_blk.at[j]], rows, sem).wait()
            if j + 1 < IDXBLK:
                pltpu.async_copy(ht_hbm.at[sidx_blk.at[j + 1]], nrows, nsem)
            pltpu.sync_copy(rows, acc.at[didx_blk.at[j]], add=True)
        return carry

    lax.fori_loop(0, NCHUNK // IDXBLK, body, 0)
    plsc.subcore_barrier()

    pltpu.sync_copy(
        acc.at[pl.ds(s * ROWS_PT, ROWS_PT)],
        out_hbm.at[c, pl.ds(s * ROWS_PT, ROWS_PT)],
    )


def _t0_body(xp_ref, w1_ref, h_ref):
    h_ref[...] = jnp.dot(xp_ref[...], w1_ref[...],
                         preferred_element_type=jnp.float32)


def _t1_body(h_ref, cnt_ref, ht_ref, dinv_ref):
    cnt = cnt_ref[0, :, 0:1] + cnt_ref[1, :, 0:1]
    deg = cnt + 1.0
    row = lax.broadcasted_iota(jnp.int32, (NP, 1), 0)
    dinv = jnp.where(row < N, lax.rsqrt(deg), 0.0)
    ht_ref[...] = h_ref[...] * dinv
    dinv_ref[...] = dinv


def _t2_body(p_ref, ht1_ref, dinv_ref, b1_ref, g_ref, be_ref, w2_ref, ht2_ref):
    agg = p_ref[0] + p_ref[1] + ht1_ref[...]
    o1 = agg * dinv_ref[...] + b1_ref[...]
    o1r = o1[:N]
    mean = jnp.mean(o1r, axis=0, keepdims=True)
    var = jnp.mean((o1r - mean) ** 2, axis=0, keepdims=True)
    h2 = jnp.maximum((o1 - mean) * lax.rsqrt(var + EPS) * g_ref[...] + be_ref[...], 0.0)
    ht2 = jnp.dot(h2, w2_ref[...], preferred_element_type=jnp.float32)
    ht2_ref[...] = ht2 * dinv_ref[...]


def _t3_body(p_ref, ht2_ref, dinv_ref, b2_ref, out_ref):
    agg = p_ref[0, :N] + p_ref[1, :N] + ht2_ref[:N]
    out_ref[...] = agg * dinv_ref[:N] + b2_ref[...]


_t0 = pl.pallas_call(
    _t0_body,
    out_shape=jax.ShapeDtypeStruct((NP, D), jnp.float32),
)

_t1 = pl.pallas_call(
    _t1_body,
    out_shape=(
        jax.ShapeDtypeStruct((NP, D), jnp.float32),
        jax.ShapeDtypeStruct((NP, 1), jnp.float32),
    ),
)

_t2 = pl.pallas_call(
    _t2_body,
    out_shape=jax.ShapeDtypeStruct((NP, D), jnp.float32),
)

_t3 = pl.pallas_call(
    _t3_body,
    out_shape=jax.ShapeDtypeStruct((N, D), jnp.float32),
)


def kernel(x, edge_index, W1, b1, gamma, beta, W2, b2):
    src = edge_index[0]
    dst = edge_index[1]
    pad = jnp.full((EPAD - E,), N, dtype=jnp.int32)
    srcp = jnp.concatenate([src, pad]).reshape(EPAD // CHUNK, CHUNK)
    dstp = jnp.concatenate([dst, pad]).reshape(EPAD // CHUNK, CHUNK)
    xp = jnp.pad(x, ((0, NP - N), (0, 0)))

    cnt = _deg_kernel(dstp)
    h1 = _t0(xp, W1)
    ht1, dinv = _t1(h1, cnt)
    p1 = _agg_kernel(ht1, srcp, dstp)
    ht2 = _t2(p1, ht1, dinv, b1.reshape(1, D), gamma.reshape(1, D),
              beta.reshape(1, D), W2)
    p2 = _agg_kernel(ht2, srcp, dstp)
    out = _t3(p2, ht2, dinv, b2.reshape(1, D))
    return out


# speedup vs baseline: 1.0651x; 1.0209x over previous
import functools

import jax
import jax.numpy as jnp
from jax import lax
from jax.experimental import pallas as pl
from jax.experimental.pallas import tpu as pltpu
from jax.experimental.pallas import tpu_sc as plsc

N = 10000
E = 320000
D = 128
EPS = 1e-5

NP = 10240
CHUNK = 128
NUM_WORKERS = 32
EPT = 10240
EPAD = EPT * NUM_WORKERS
NCHUNK = EPT // CHUNK
IDXBLK = 8
ROWS_PT = NP // 16

_mesh = plsc.VectorSubcoreMesh(core_axis_name="c", subcore_axis_name="s")


@functools.partial(
    pl.kernel,
    out_type=jax.ShapeDtypeStruct((2, NP, D), jnp.float32),
    mesh=_mesh,
    scratch_types=[
        pltpu.VMEM((NCHUNK, CHUNK), jnp.int32),
        pltpu.VMEM((CHUNK, D), jnp.float32),
        pltpu.VMEM((16, D), jnp.float32),
        pltpu.VMEM_SHARED((NP, D), jnp.float32),
    ],
)
def _deg_kernel(dst_hbm, out_hbm, didx_all, ones_v, zero_v, acc):
    c = lax.axis_index("c")
    s = lax.axis_index("s")
    wid = s * 2 + c

    pltpu.sync_copy(dst_hbm.at[pl.ds(wid * NCHUNK, NCHUNK)], didx_all)

    for i in range(16):
        for j in range(D // 16):
            zero_v[i, pl.ds(j * 16, 16)] = jnp.zeros((16,), jnp.float32)
    for i in range(CHUNK):
        for j in range(D // 16):
            ones_v[i, pl.ds(j * 16, 16)] = jnp.ones((16,), jnp.float32)

    for k in range(ROWS_PT // 16):
        pltpu.sync_copy(zero_v, acc.at[pl.ds(s * ROWS_PT + k * 16, 16)])
    plsc.subcore_barrier()

    def body(g, carry):
        pltpu.sync_copy(ones_v, acc.at[didx_all.at[g]], add=True)
        return carry

    lax.fori_loop(0, NCHUNK, body, 0)
    plsc.subcore_barrier()

    pltpu.sync_copy(
        acc.at[pl.ds(s * ROWS_PT, ROWS_PT)],
        out_hbm.at[c, pl.ds(s * ROWS_PT, ROWS_PT)],
    )


@functools.partial(
    pl.kernel,
    out_type=jax.ShapeDtypeStruct((2, NP, D), jnp.float32),
    mesh=_mesh,
    scratch_types=[
        pltpu.VMEM((IDXBLK, CHUNK), jnp.int32),
        pltpu.VMEM((IDXBLK, CHUNK), jnp.int32),
        pltpu.VMEM((IDXBLK, CHUNK), jnp.int32),
        pltpu.VMEM((IDXBLK, CHUNK), jnp.int32),
        pltpu.VMEM((CHUNK, D), jnp.float32),
        pltpu.VMEM((CHUNK, D), jnp.float32),
        pltpu.VMEM((16, D), jnp.float32),
        pltpu.VMEM_SHARED((NP, D), jnp.float32),
        pltpu.SemaphoreType.DMA,
        pltpu.SemaphoreType.DMA,
        pltpu.SemaphoreType.DMA,
        pltpu.SemaphoreType.DMA,
        pltpu.SemaphoreType.DMA,
        pltpu.SemaphoreType.DMA,
    ],
)
def _agg_kernel(ht_hbm, src_hbm, dst_hbm, out_hbm, sidxA, didxA, sidxB, didxB,
                rows0, rows1, zero_v, acc, sem0, sem1, ssA, dsA, ssB, dsB):
    c = lax.axis_index("c")
    s = lax.axis_index("s")
    wid = s * 2 + c

    for i in range(16):
        for j in range(D // 16):
            zero_v[i, pl.ds(j * 16, 16)] = jnp.zeros((16,), jnp.float32)

    for k in range(ROWS_PT // 16):
        pltpu.sync_copy(zero_v, acc.at[pl.ds(s * ROWS_PT + k * 16, 16)])
    plsc.subcore_barrier()

    def stage(b, ss, ds_, ssem, dsem):
        base = wid * NCHUNK + b * IDXBLK
        pltpu.async_copy(src_hbm.at[pl.ds(base, IDXBLK)], ss, ssem)
        pltpu.async_copy(dst_hbm.at[pl.ds(base, IDXBLK)], ds_, dsem)

    def wait_stage(b, ss, ds_, ssem, dsem):
        base = wid * NCHUNK + b * IDXBLK
        pltpu.make_async_copy(src_hbm.at[pl.ds(base, IDXBLK)], ss, ssem).wait()
        pltpu.make_async_copy(dst_hbm.at[pl.ds(base, IDXBLK)], ds_, dsem).wait()

    def block(b, cs, cd, cssem, cdsem, nxt, prefetch, stage_b):
        for j in range(IDXBLK):
            rows, sem = (rows0, sem0) if j % 2 == 0 else (rows1, sem1)
            nrows, nsem = (rows1, sem1) if j % 2 == 0 else (rows0, sem0)
            pltpu.make_async_copy(ht_hbm.at[cs.at[j]], rows, sem).wait()
            if j + 1 < IDXBLK:
                pltpu.async_copy(ht_hbm.at[cs.at[j + 1]], nrows, nsem)
            elif prefetch:
                ns, nd, nssem, ndsem = nxt
                wait_stage(b + 1, ns, nd, nssem, ndsem)
                pltpu.async_copy(ht_hbm.at[ns.at[0]], nrows, nsem)
            pltpu.sync_copy(rows, acc.at[cd.at[j]], add=True)
        if stage_b is not None:
            stage(stage_b, cs, cd, cssem, cdsem)

    base0 = wid * NCHUNK
    pltpu.sync_copy(src_hbm.at[pl.ds(base0, IDXBLK)], sidxA)
    pltpu.sync_copy(dst_hbm.at[pl.ds(base0, IDXBLK)], didxA)
    pltpu.async_copy(ht_hbm.at[sidxA.at[0]], rows0, sem0)
    stage(1, sidxB, didxB, ssB, dsB)

    def pair(p, carry):
        b0 = 2 * p
        block(b0, sidxA, didxA, ssA, dsA, (sidxB, didxB, ssB, dsB), True, b0 + 2)
        block(b0 + 1, sidxB, didxB, ssB, dsB, (sidxA, didxA, ssA, dsA), True, b0 + 3)
        return carry

    lax.fori_loop(0, NCHUNK // IDXBLK // 2 - 1, pair, 0)

    last = NCHUNK // IDXBLK - 2
    block(last, sidxA, didxA, ssA, dsA, (sidxB, didxB, ssB, dsB), True, None)
    block(last + 1, sidxB, didxB, ssB, dsB, (sidxA, didxA, ssA, dsA), False, None)
    plsc.subcore_barrier()

    pltpu.sync_copy(
        acc.at[pl.ds(s * ROWS_PT, ROWS_PT)],
        out_hbm.at[c, pl.ds(s * ROWS_PT, ROWS_PT)],
    )


def _t0_body(xp_ref, w1_ref, h_ref):
    h_ref[...] = jnp.dot(xp_ref[...], w1_ref[...],
                         preferred_element_type=jnp.float32)


def _t1_body(h_ref, cnt_ref, ht_ref, dinv_ref):
    cnt = cnt_ref[0, :, 0:1] + cnt_ref[1, :, 0:1]
    deg = cnt + 1.0
    row = lax.broadcasted_iota(jnp.int32, (NP, 1), 0)
    dinv = jnp.where(row < N, lax.rsqrt(deg), 0.0)
    ht_ref[...] = h_ref[...] * dinv
    dinv_ref[...] = dinv


def _t2_body(p_ref, ht1_ref, dinv_ref, b1_ref, g_ref, be_ref, w2_ref, ht2_ref):
    agg = p_ref[0] + p_ref[1] + ht1_ref[...]
    o1 = agg * dinv_ref[...] + b1_ref[...]
    o1r = o1[:N]
    mean = jnp.mean(o1r, axis=0, keepdims=True)
    var = jnp.mean((o1r - mean) ** 2, axis=0, keepdims=True)
    h2 = jnp.maximum((o1 - mean) * lax.rsqrt(var + EPS) * g_ref[...] + be_ref[...], 0.0)
    ht2 = jnp.dot(h2, w2_ref[...], preferred_element_type=jnp.float32)
    ht2_ref[...] = ht2 * dinv_ref[...]


def _t3_body(p_ref, ht2_ref, dinv_ref, b2_ref, out_ref):
    agg = p_ref[0, :N] + p_ref[1, :N] + ht2_ref[:N]
    out_ref[...] = agg * dinv_ref[:N] + b2_ref[...]


_t0 = pl.pallas_call(
    _t0_body,
    out_shape=jax.ShapeDtypeStruct((NP, D), jnp.float32),
)

_t1 = pl.pallas_call(
    _t1_body,
    out_shape=(
        jax.ShapeDtypeStruct((NP, D), jnp.float32),
        jax.ShapeDtypeStruct((NP, 1), jnp.float32),
    ),
)

_t2 = pl.pallas_call(
    _t2_body,
    out_shape=jax.ShapeDtypeStruct((NP, D), jnp.float32),
)

_t3 = pl.pallas_call(
    _t3_body,
    out_shape=jax.ShapeDtypeStruct((N, D), jnp.float32),
)


def kernel(x, edge_index, W1, b1, gamma, beta, W2, b2):
    src = edge_index[0]
    dst = edge_index[1]
    pad = jnp.full((EPAD - E,), N, dtype=jnp.int32)
    srcp = jnp.concatenate([src, pad]).reshape(EPAD // CHUNK, CHUNK)
    dstp = jnp.concatenate([dst, pad]).reshape(EPAD // CHUNK, CHUNK)
    xp = jnp.pad(x, ((0, NP - N), (0, 0)))

    cnt = _deg_kernel(dstp)
    h1 = _t0(xp, W1)
    ht1, dinv = _t1(h1, cnt)
    p1 = _agg_kernel(ht1, srcp, dstp)
    ht2 = _t2(p1, ht1, dinv, b1.reshape(1, D), gamma.reshape(1, D),
              beta.reshape(1, D), W2)
    p2 = _agg_kernel(ht2, srcp, dstp)
    out = _t3(p2, ht2, dinv, b2.reshape(1, D))
    return out
